# baseline (speedup 1.0000x reference)
import functools

import jax
import jax.numpy as jnp
from jax import lax
from jax.experimental import pallas as pl
from jax.experimental.pallas import tpu as pltpu
from jax.experimental.pallas import tpu_sc as plsc

B = 16
C = 2
HW = 512 * 512
K = 500
KPAD = 512
LANES = 16
KH = KPAD
NCHUNK = KH // LANES
ELEMS_PER_TILE = 2 * KH
GATHER_GROUP = 128
NGROUP = ELEMS_PER_TILE // GATHER_GROUP


def _sc_body(table_hbm, ind_hbm, tgt_hbm,
             part_out,
             ind_v, tgt_v, idx_v, vals_v, acc_v, sem, sem2, sem3):
    wid = lax.axis_index("s")
    b = wid
    k0 = 0

    tcp = pltpu.async_copy(tgt_hbm.at[b], tgt_v, sem2)
    pltpu.sync_copy(ind_hbm.at[b, pl.ds(k0, KH)], ind_v)

    base0 = b * (C * HW)

    def _idx_body(li, carry):
        raw = ind_v[pl.ds(pl.multiple_of(li * LANES, LANES), LANES)]
        ind_c = jnp.maximum(raw, 0)
        flat0 = (base0 + (ind_c & -4096)
                 + ((ind_c & (3 << 7)) << 3)
                 + ((ind_c & (7 << 9)) >> 2)
                 + (ind_c & 127))
        g = li // 8
        o = pl.multiple_of((li % 8) * LANES, LANES)
        idx_v[g, pl.ds(o, LANES)] = flat0
        idx_v[NGROUP // 2 + g, pl.ds(o, LANES)] = flat0 + HW
        return carry

    def _fire(j, s):
        return pltpu.async_copy(
            table_hbm.at[idx_v.at[j]],
            vals_v.at[pl.ds(j * GATHER_GROUP, GATHER_GROUP)],
            s,
        )

    lax.fori_loop(0, NCHUNK // 2, _idx_body, 0)
    ga = [_fire(0, sem), _fire(1, sem), _fire(4, sem), _fire(5, sem)]
    lax.fori_loop(NCHUNK // 2, NCHUNK, _idx_body, 0)
    gb = [_fire(2, sem3), _fire(3, sem3), _fire(6, sem3), _fire(7, sem3)]
    tcp.wait()

    zero = jnp.zeros((LANES,), jnp.float32)

    def _loss_body(li, carry):
        l_acc, n_acc = carry
        lsl = pl.ds(pl.multiple_of(li * LANES, LANES), LANES)
        v0 = vals_v[lsl]
        v1 = vals_v[pl.ds(pl.multiple_of(li * LANES + KH, LANES), LANES)]
        m = jnp.where(ind_v[lsl] >= 0, 1.0, 0.0)
        tbase = (li >> 3) * 256 + (li & 7) * LANES
        t0 = tgt_v[pl.ds(pl.multiple_of(tbase, LANES), LANES)]
        t1 = tgt_v[pl.ds(pl.multiple_of(tbase + 128, LANES), LANES)]
        d0 = (v0 - t0) * m
        d1 = (v1 - t1) * m
        a0 = jnp.abs(d0)
        a1 = jnp.abs(d1)
        e0 = jnp.where(a0 < 1.0, 0.5 * d0 * d0, a0 - 0.5)
        e1 = jnp.where(a1 < 1.0, 0.5 * d1 * d1, a1 - 0.5)
        return l_acc + e0 + e1, n_acc + m

    for cp in ga:
        cp.wait()
    acc_half = lax.fori_loop(0, NCHUNK // 2, _loss_body, (zero, zero))
    for cp in gb:
        cp.wait()
    loss_acc, num_acc = lax.fori_loop(
        NCHUNK // 2, NCHUNK, _loss_body, acc_half)

    acc_v[pl.ds(0, LANES)] = loss_acc
    acc_v[pl.ds(LANES, LANES)] = num_acc
    for z in range(2, 8):
        acc_v[pl.ds(z * LANES, LANES)] = zero
    pltpu.sync_copy(acc_v, part_out.at[wid])


@jax.jit
def _reg_loss_sc(table, ind_m, tgt4):
    mesh = plsc.VectorSubcoreMesh(core_axis_name="c", subcore_axis_name="s", num_cores=1)
    k = functools.partial(
        pl.kernel,
        mesh=mesh,
        out_type=jax.ShapeDtypeStruct((16, 128), jnp.float32),
        scratch_types=[
            pltpu.VMEM((KH,), jnp.int32),
            pltpu.VMEM((2 * KH,), jnp.float32),
            pltpu.VMEM((NGROUP, GATHER_GROUP), jnp.int32),
            pltpu.VMEM((ELEMS_PER_TILE,), jnp.float32),
            pltpu.VMEM((128,), jnp.float32),
            pltpu.SemaphoreType.DMA,
            pltpu.SemaphoreType.DMA,
            pltpu.SemaphoreType.DMA,
        ],
    )(_sc_body)
    part = k(table, ind_m, tgt4)
    num = part[:, LANES:2 * LANES].sum()
    return part[:, :LANES].sum() / (num + 0.0001)


def kernel(output, mask, ind, target):
    table = jnp.transpose(
        output.reshape(B, C, 512 // 8, 8, 512 // 128, 128),
        (0, 1, 2, 4, 3, 5)).reshape(B * C * HW)
    ind_m = jnp.pad(jnp.where(mask, ind.astype(jnp.int32), -1),
                    ((0, 0), (0, KPAD - K)), constant_values=-1)
    t4 = jnp.transpose(
        jnp.pad(target, ((0, 0), (0, KPAD - K), (0, 0))).reshape(
            B, KPAD // 128, 128, C),
        (0, 1, 3, 2)).reshape(B, 2 * KPAD)
    return _reg_loss_sc(table, ind_m, t4)


# speedup vs baseline: 1.0286x; 1.0286x over previous
import functools

import jax
import jax.numpy as jnp
from jax import lax
from jax.experimental import pallas as pl
from jax.experimental.pallas import tpu as pltpu
from jax.experimental.pallas import tpu_sc as plsc

B = 16
C = 2
HW = 512 * 512
K = 500
KPAD = 512
LANES = 16
KH = KPAD
NCHUNK = KH // LANES
ELEMS_PER_TILE = 2 * KH
GATHER_GROUP = 128
NGROUP = ELEMS_PER_TILE // GATHER_GROUP


def _sc_body(table_hbm, ind_hbm, buf_hbm,
             part_out,
             ind_v, mask_v, tgt_v, idx_v, vals_v, acc_v, sem, sem2, sem3):
    wid = lax.axis_index("s")
    b = wid
    k0 = 0

    mcp = pltpu.async_copy(buf_hbm.at[b, pl.ds(k0, KH)], mask_v, sem2)
    tcp = pltpu.async_copy(
        buf_hbm.at[b, pl.ds(KPAD, 2 * KH)], tgt_v, sem2)
    pltpu.sync_copy(ind_hbm.at[b, pl.ds(k0, KH)], ind_v)

    base0 = b * (C * HW)

    def _idx_body(li, carry):
        ind_c = ind_v[pl.ds(pl.multiple_of(li * LANES, LANES), LANES)]
        flat0 = (base0 + (ind_c & -4096)
                 + ((ind_c & (3 << 7)) << 3)
                 + ((ind_c & (7 << 9)) >> 2)
                 + (ind_c & 127))
        g = li // 8
        o = pl.multiple_of((li % 8) * LANES, LANES)
        idx_v[g, pl.ds(o, LANES)] = flat0
        idx_v[NGROUP // 2 + g, pl.ds(o, LANES)] = flat0 + HW
        return carry

    def _fire(j, s):
        return pltpu.async_copy(
            table_hbm.at[idx_v.at[j]],
            vals_v.at[pl.ds(j * GATHER_GROUP, GATHER_GROUP)],
            s,
        )

    lax.fori_loop(0, NCHUNK // 2, _idx_body, 0)
    ga = [_fire(0, sem), _fire(1, sem), _fire(4, sem), _fire(5, sem)]
    lax.fori_loop(NCHUNK // 2, NCHUNK, _idx_body, 0)
    gb = [_fire(2, sem3), _fire(3, sem3), _fire(6, sem3), _fire(7, sem3)]
    mcp.wait()
    tcp.wait()

    zero = jnp.zeros((LANES,), jnp.float32)

    def _loss_body(li, carry):
        l_acc, n_acc = carry
        lsl = pl.ds(pl.multiple_of(li * LANES, LANES), LANES)
        v0 = vals_v[lsl]
        v1 = vals_v[pl.ds(pl.multiple_of(li * LANES + KH, LANES), LANES)]
        m = mask_v[lsl]
        tbase = (li >> 3) * 256 + (li & 7) * LANES
        t0 = tgt_v[pl.ds(pl.multiple_of(tbase, LANES), LANES)]
        t1 = tgt_v[pl.ds(pl.multiple_of(tbase + 128, LANES), LANES)]
        d0 = (v0 - t0) * m
        d1 = (v1 - t1) * m
        a0 = jnp.abs(d0)
        a1 = jnp.abs(d1)
        e0 = jnp.where(a0 < 1.0, 0.5 * d0 * d0, a0 - 0.5)
        e1 = jnp.where(a1 < 1.0, 0.5 * d1 * d1, a1 - 0.5)
        return l_acc + e0 + e1, n_acc + m

    for cp in ga:
        cp.wait()
    acc_half = lax.fori_loop(0, NCHUNK // 2, _loss_body, (zero, zero))
    for cp in gb:
        cp.wait()
    loss_acc, num_acc = lax.fori_loop(
        NCHUNK // 2, NCHUNK, _loss_body, acc_half)

    acc_v[pl.ds(0, LANES)] = loss_acc
    acc_v[pl.ds(LANES, LANES)] = num_acc
    for z in range(2, 8):
        acc_v[pl.ds(z * LANES, LANES)] = zero
    pltpu.sync_copy(acc_v, part_out.at[wid])


@jax.jit
def _reg_loss_sc(table, ind_pad, buf):
    mesh = plsc.VectorSubcoreMesh(core_axis_name="c", subcore_axis_name="s", num_cores=1)
    k = functools.partial(
        pl.kernel,
        mesh=mesh,
        out_type=jax.ShapeDtypeStruct((16, 128), jnp.float32),
        scratch_types=[
            pltpu.VMEM((KH,), jnp.int32),
            pltpu.VMEM((KH,), jnp.float32),
            pltpu.VMEM((2 * KH,), jnp.float32),
            pltpu.VMEM((NGROUP, GATHER_GROUP), jnp.int32),
            pltpu.VMEM((ELEMS_PER_TILE,), jnp.float32),
            pltpu.VMEM((128,), jnp.float32),
            pltpu.SemaphoreType.DMA,
            pltpu.SemaphoreType.DMA,
            pltpu.SemaphoreType.DMA,
        ],
    )(_sc_body)
    part = k(table, ind_pad, buf)
    num = part[:, LANES:2 * LANES].sum()
    return part[:, :LANES].sum() / (num + 0.0001)


def kernel(output, mask, ind, target):
    table = jnp.transpose(
        output.reshape(B, C, 512 // 8, 8, 512 // 128, 128),
        (0, 1, 2, 4, 3, 5)).reshape(B * C * HW)
    ind_pad = jnp.pad(ind.astype(jnp.int32), ((0, 0), (0, KPAD - K)))
    maskf = jnp.pad(mask.astype(jnp.float32), ((0, 0), (0, KPAD - K)))
    t4 = jnp.transpose(
        jnp.pad(target, ((0, 0), (0, KPAD - K), (0, 0))).reshape(
            B, KPAD // 128, 128, C),
        (0, 1, 3, 2)).reshape(B, 2 * KPAD)
    buf = jnp.concatenate([maskf, t4], axis=1)
    return _reg_loss_sc(table, ind_pad, buf)


# speedup vs baseline: 1.0294x; 1.0007x over previous
import functools

import jax
import jax.numpy as jnp
from jax import lax
from jax.experimental import pallas as pl
from jax.experimental.pallas import tpu as pltpu
from jax.experimental.pallas import tpu_sc as plsc

B = 16
C = 2
HW = 512 * 512
K = 500
KPAD = 512
LANES = 16
KH = KPAD
NCHUNK = KH // LANES
ELEMS_PER_TILE = 2 * KH
GATHER_GROUP = 128
NGROUP = ELEMS_PER_TILE // GATHER_GROUP


def _sc_body(table_hbm, ind_hbm, buf_hbm,
             part_out,
             ind_v, mask_v, tgt_v, idx_v, vals_v, acc_v, sem, sem2, sem3):
    wid = lax.axis_index("s")
    b = wid
    k0 = 0

    mcp = pltpu.async_copy(buf_hbm.at[b, pl.ds(k0, KH)], mask_v, sem2)
    tcp = pltpu.async_copy(
        buf_hbm.at[b, pl.ds(KPAD, 2 * KH)], tgt_v, sem2)
    icp = [
        pltpu.async_copy(
            ind_hbm.at[(b // 8) * 32 + kt * 8 + (b % 8)],
            ind_v.at[pl.ds(kt * 128, 128)], sem3)
        for kt in range(4)
    ]
    for cp in icp:
        cp.wait()

    base0 = b * (C * HW)

    def _idx_body(li, carry):
        ind_c = ind_v[pl.ds(pl.multiple_of(li * LANES, LANES), LANES)]
        flat0 = (base0 + (ind_c & -4096)
                 + ((ind_c & (3 << 7)) << 3)
                 + ((ind_c & (7 << 9)) >> 2)
                 + (ind_c & 127))
        g = li // 8
        o = pl.multiple_of((li % 8) * LANES, LANES)
        idx_v[g, pl.ds(o, LANES)] = flat0
        idx_v[NGROUP // 2 + g, pl.ds(o, LANES)] = flat0 + HW
        return carry

    def _fire(j, s):
        return pltpu.async_copy(
            table_hbm.at[idx_v.at[j]],
            vals_v.at[pl.ds(j * GATHER_GROUP, GATHER_GROUP)],
            s,
        )

    lax.fori_loop(0, NCHUNK // 2, _idx_body, 0)
    ga = [_fire(0, sem), _fire(1, sem), _fire(4, sem), _fire(5, sem)]
    lax.fori_loop(NCHUNK // 2, NCHUNK, _idx_body, 0)
    gb = [_fire(2, sem3), _fire(3, sem3), _fire(6, sem3), _fire(7, sem3)]
    mcp.wait()
    tcp.wait()

    zero = jnp.zeros((LANES,), jnp.float32)

    def _loss_body(li, carry):
        l_acc, n_acc = carry
        lsl = pl.ds(pl.multiple_of(li * LANES, LANES), LANES)
        v0 = vals_v[lsl]
        v1 = vals_v[pl.ds(pl.multiple_of(li * LANES + KH, LANES), LANES)]
        m = mask_v[lsl]
        tbase = (li >> 3) * 256 + (li & 7) * LANES
        t0 = tgt_v[pl.ds(pl.multiple_of(tbase, LANES), LANES)]
        t1 = tgt_v[pl.ds(pl.multiple_of(tbase + 128, LANES), LANES)]
        d0 = (v0 - t0) * m
        d1 = (v1 - t1) * m
        a0 = jnp.abs(d0)
        a1 = jnp.abs(d1)
        e0 = jnp.where(a0 < 1.0, 0.5 * d0 * d0, a0 - 0.5)
        e1 = jnp.where(a1 < 1.0, 0.5 * d1 * d1, a1 - 0.5)
        return l_acc + e0 + e1, n_acc + m

    for cp in ga:
        cp.wait()
    acc_half = lax.fori_loop(0, NCHUNK // 2, _loss_body, (zero, zero))
    for cp in gb:
        cp.wait()
    loss_acc, num_acc = lax.fori_loop(
        NCHUNK // 2, NCHUNK, _loss_body, acc_half)

    acc_v[pl.ds(0, LANES)] = loss_acc
    acc_v[pl.ds(LANES, LANES)] = num_acc
    for z in range(2, 8):
        acc_v[pl.ds(z * LANES, LANES)] = zero
    pltpu.sync_copy(acc_v, part_out.at[wid])


@jax.jit
def _reg_loss_sc(table, ind_pad, buf):
    mesh = plsc.VectorSubcoreMesh(core_axis_name="c", subcore_axis_name="s", num_cores=1)
    k = functools.partial(
        pl.kernel,
        mesh=mesh,
        out_type=jax.ShapeDtypeStruct((16, 128), jnp.float32),
        scratch_types=[
            pltpu.VMEM((KH,), jnp.int32),
            pltpu.VMEM((KH,), jnp.float32),
            pltpu.VMEM((2 * KH,), jnp.float32),
            pltpu.VMEM((NGROUP, GATHER_GROUP), jnp.int32),
            pltpu.VMEM((ELEMS_PER_TILE,), jnp.float32),
            pltpu.VMEM((128,), jnp.float32),
            pltpu.SemaphoreType.DMA,
            pltpu.SemaphoreType.DMA,
            pltpu.SemaphoreType.DMA,
        ],
    )(_sc_body)
    part = k(table, ind_pad, buf)
    num = part[:, LANES:2 * LANES].sum()
    return part[:, :LANES].sum() / (num + 0.0001)


def kernel(output, mask, ind, target):
    table = jnp.transpose(
        output.reshape(B, C, 512 // 8, 8, 512 // 128, 128),
        (0, 1, 2, 4, 3, 5)).reshape(B * C * HW)
    ind_pad = jnp.transpose(
        jnp.pad(ind.astype(jnp.int32), ((0, 0), (0, KPAD - K))).reshape(
            2, 8, 4, 128),
        (0, 2, 1, 3)).reshape(64, 128)
    maskf = jnp.pad(mask.astype(jnp.float32), ((0, 0), (0, KPAD - K)))
    t4 = jnp.transpose(
        jnp.pad(target, ((0, 0), (0, KPAD - K), (0, 0))).reshape(
            B, KPAD // 128, 128, C),
        (0, 1, 3, 2)).reshape(B, 2 * KPAD)
    buf = jnp.concatenate([maskf, t4], axis=1)
    return _reg_loss_sc(table, ind_pad, buf)
